# baseline (speedup 1.0000x reference)
import functools

import jax
import jax.numpy as jnp
import numpy as np
from jax import lax
from jax.experimental import pallas as pl
from jax.experimental.pallas import tpu as pltpu
from jax.experimental.pallas import tpu_sc as plsc

_N = 262144
_K = 16
_T = 500
_TPAD = 512
_NC, _NS = 2, 16
_NW = _NC * _NS
_BPW = _N // _NW
_B = 1024


def _build_r48() -> np.ndarray:
    r = np.zeros((48, 512), np.float32)
    for i in range(_K):
        for k in range(_K):
            c = i * _K + k
            r[k, c] = 1.0
            r[16 + i, 256 + c] = 1.0
            if i == k:
                r[32 + i, 256 + c] = 1.0
    return r


_R48 = _build_r48()


def _sc_gather(tbl_flat, t):
    mesh = plsc.VectorSubcoreMesh(
        core_axis_name="c", subcore_axis_name="s",
        num_cores=_NC, num_subcores=_NS)

    @functools.partial(
        pl.kernel,
        out_type=jax.ShapeDtypeStruct((8, _N), jnp.float32),
        mesh=mesh,
        compiler_params=pltpu.CompilerParams(needs_layout_passes=False),
        scratch_types=[
            pltpu.VMEM((8 * _TPAD,), jnp.float32),
            pltpu.VMEM((_BPW,), jnp.int32),
            pltpu.VMEM((8, _BPW), jnp.float32),
        ],
    )
    def gather_kernel(tbl_hbm, t_hbm, out_hbm, tbl_v, idx_v, buf_v):
        wid = lax.axis_index("s") * _NC + lax.axis_index("c")
        base = wid * _BPW
        pltpu.sync_copy(tbl_hbm, tbl_v)
        pltpu.sync_copy(t_hbm.at[pl.ds(base, _BPW)], idx_v)

        def body(g, carry):
            off = g * 16
            tvec = idx_v[pl.ds(off, 16)]
            for c in range(8):
                v = plsc.load_gather(tbl_v, [tvec + (c * _TPAD)])
                buf_v[c, pl.ds(off, 16)] = v
            return carry

        lax.fori_loop(0, _BPW // 16, body, 0)
        pltpu.sync_copy(buf_v, out_hbm.at[:, pl.ds(base, _BPW)])

    return gather_kernel(tbl_flat, t)


def _tc_body(xtT_ref, scT_ref, r48_ref, out_ref):
    xtT = xtT_ref[...]
    sc = scT_ref[...]
    s_sum = jnp.sum(xtT, axis=0, keepdims=True)
    qs, qd = sc[0:1], sc[1:2]
    bs, bd = sc[2:3], sc[3:4]
    cs, cd = sc[4:5], sc[5:6]
    a = qd * s_sum + (qs - qd) * xtT
    p1 = cd * s_sum + (cs - cd) * xtT
    r = 1.0 / jnp.maximum(p1, 1e-5)
    u = bd * r
    w = (bs - bd) * r
    cat = jnp.concatenate([a, u, w], axis=0)
    res = lax.dot_general(
        cat, r48_ref[...], (((0,), (0,)), ((), ())),
        preferred_element_type=jnp.float32)
    out_ref[...] = res[:, :256] * res[:, 256:]


def _tc_dense(xtT, scT, n):
    return pl.pallas_call(
        _tc_body,
        grid=(n // _B,),
        in_specs=[
            pl.BlockSpec((_K, _B), lambda i: (0, i)),
            pl.BlockSpec((8, _B), lambda i: (0, i)),
            pl.BlockSpec((48, 512), lambda i: (0, 0)),
        ],
        out_specs=pl.BlockSpec((_B, 256), lambda i: (i, 0)),
        out_shape=jax.ShapeDtypeStruct((n, 256), jnp.float32),
    )(xtT, scT, jnp.asarray(_R48))


def kernel(xt, t, Qt, Qt_bar, Qt_bar_prev):
    _SKIP_RESHAPE = True
    n = xt.shape[0]
    tbl6 = jnp.stack(
        [Qt[:, 0, 0], Qt[:, 0, 1],
         Qt_bar_prev[:, 0, 0], Qt_bar_prev[:, 0, 1],
         Qt_bar[:, 0, 0], Qt_bar[:, 0, 1]], axis=1)
    tbl = jnp.zeros((8, _TPAD), jnp.float32).at[:6, :_T].set(tbl6.T)
    scT = _sc_gather(tbl.reshape(-1), t)
    xtT = xt.T
    out = _tc_dense(xtT, scT, n)
    if _SKIP_RESHAPE:
        return out
    return out.reshape(n, _K, _K)


# speedup vs baseline: 1.0001x; 1.0001x over previous
import functools

import jax
import jax.numpy as jnp
import numpy as np
from jax import lax
from jax.experimental import pallas as pl
from jax.experimental.pallas import tpu as pltpu
from jax.experimental.pallas import tpu_sc as plsc

_N = 262144
_K = 16
_T = 500
_TPAD = 512
_NC, _NS = 2, 16
_NW = _NC * _NS
_BPW = _N // _NW
_B = 1024


def _build_r48() -> np.ndarray:
    r = np.zeros((48, 512), np.float32)
    for i in range(_K):
        for k in range(_K):
            c = i * _K + k
            r[k, c] = 1.0
            r[16 + i, 256 + c] = 1.0
            if i == k:
                r[32 + i, 256 + c] = 1.0
    return r


_R48 = _build_r48()


def _sc_gather(tbl_flat, t):
    mesh = plsc.VectorSubcoreMesh(
        core_axis_name="c", subcore_axis_name="s",
        num_cores=_NC, num_subcores=_NS)

    @functools.partial(
        pl.kernel,
        out_type=jax.ShapeDtypeStruct((8, _N), jnp.float32),
        mesh=mesh,
        compiler_params=pltpu.CompilerParams(needs_layout_passes=False),
        scratch_types=[
            pltpu.VMEM((8 * _TPAD,), jnp.float32),
            pltpu.VMEM((_BPW,), jnp.int32),
            pltpu.VMEM((8, _BPW), jnp.float32),
        ],
    )
    def gather_kernel(tbl_hbm, t_hbm, out_hbm, tbl_v, idx_v, buf_v):
        wid = lax.axis_index("s") * _NC + lax.axis_index("c")
        base = wid * _BPW
        pltpu.sync_copy(tbl_hbm, tbl_v)
        pltpu.sync_copy(t_hbm.at[pl.ds(base, _BPW)], idx_v)

        def body(g, carry):
            off = g * 16
            tvec = idx_v[pl.ds(off, 16)]
            for c in range(8):
                v = plsc.load_gather(tbl_v, [tvec + (c * _TPAD)])
                buf_v[c, pl.ds(off, 16)] = v
            return carry

        lax.fori_loop(0, _BPW // 16, body, 0)
        pltpu.sync_copy(buf_v, out_hbm.at[:, pl.ds(base, _BPW)])

    return gather_kernel(tbl_flat, t)


def _tc_body(xtT_ref, scT_ref, l48_ref, out_ref):
    xtT = xtT_ref[...]
    sc = scT_ref[...]
    s_sum = jnp.sum(xtT, axis=0, keepdims=True)
    qs, qd = sc[0:1], sc[1:2]
    bs, bd = sc[2:3], sc[3:4]
    cs, cd = sc[4:5], sc[5:6]
    a = qd * s_sum + (qs - qd) * xtT
    p1 = cd * s_sum + (cs - cd) * xtT
    r = 1.0 / jnp.maximum(p1, 1e-5)
    u = bd * r
    w = (bs - bd) * r
    cat = jnp.concatenate([a, u, w], axis=0)
    res = lax.dot_general(
        l48_ref[...], cat, (((1,), (0,)), ((), ())),
        preferred_element_type=jnp.float32)
    out_ref[...] = res[:256] * res[256:]


def _tc_dense(xtT, scT, n):
    return pl.pallas_call(
        _tc_body,
        grid=(n // _B,),
        in_specs=[
            pl.BlockSpec((_K, _B), lambda i: (0, i)),
            pl.BlockSpec((8, _B), lambda i: (0, i)),
            pl.BlockSpec((512, 48), lambda i: (0, 0)),
        ],
        out_specs=pl.BlockSpec((256, _B), lambda i: (0, i)),
        out_shape=jax.ShapeDtypeStruct((256, n), jnp.float32),
    )(xtT, scT, jnp.asarray(_R48.T))


def kernel(xt, t, Qt, Qt_bar, Qt_bar_prev):
    n = xt.shape[0]
    tbl6 = jnp.stack(
        [Qt[:, 0, 0], Qt[:, 0, 1],
         Qt_bar_prev[:, 0, 0], Qt_bar_prev[:, 0, 1],
         Qt_bar[:, 0, 0], Qt_bar[:, 0, 1]], axis=1)
    tbl = jnp.zeros((8, _TPAD), jnp.float32).at[:6, :_T].set(tbl6.T)
    scT = _sc_gather(tbl.reshape(-1), t)
    xtT = xt.T
    out = _tc_dense(xtT, scT, n)
    return out.reshape(_K, _K, n).transpose(2, 0, 1)


# speedup vs baseline: 1.3496x; 1.3494x over previous
import functools

import jax
import jax.numpy as jnp
import numpy as np
from jax import lax
from jax.experimental import pallas as pl
from jax.experimental.pallas import tpu as pltpu
from jax.experimental.pallas import tpu_sc as plsc

_N = 262144
_K = 16
_T = 500
_TPAD = 512
_NC, _NS = 2, 16
_NW = _NC * _NS
_BPW = _N // _NW
_B = 2048


def _build_r48() -> np.ndarray:
    r = np.zeros((48, 512), np.float32)
    for i in range(_K):
        for k in range(_K):
            c = i * _K + k
            r[k, c] = 1.0
            r[16 + i, 256 + c] = 1.0
            if i == k:
                r[32 + i, 256 + c] = 1.0
    return r


_R48 = _build_r48()


def _sc_gather(tbl_flat, t):
    mesh = plsc.VectorSubcoreMesh(
        core_axis_name="c", subcore_axis_name="s",
        num_cores=_NC, num_subcores=_NS)

    @functools.partial(
        pl.kernel,
        out_type=jax.ShapeDtypeStruct((8, _N), jnp.float32),
        mesh=mesh,
        compiler_params=pltpu.CompilerParams(needs_layout_passes=False),
        scratch_types=[
            pltpu.VMEM((8 * _TPAD,), jnp.float32),
            pltpu.VMEM((_BPW,), jnp.int32),
            pltpu.VMEM((8, _BPW), jnp.float32),
        ],
    )
    def gather_kernel(tbl_hbm, t_hbm, out_hbm, tbl_v, idx_v, buf_v):
        wid = lax.axis_index("s") * _NC + lax.axis_index("c")
        base = wid * _BPW
        pltpu.sync_copy(tbl_hbm, tbl_v)
        pltpu.sync_copy(t_hbm.at[pl.ds(base, _BPW)], idx_v)

        def body(g, carry):
            off = g * 16
            tvec = idx_v[pl.ds(off, 16)]
            for c in range(8):
                v = plsc.load_gather(tbl_v, [tvec + (c * _TPAD)])
                buf_v[c, pl.ds(off, 16)] = v
            return carry

        lax.fori_loop(0, _BPW // 16, body, 0)
        pltpu.sync_copy(buf_v, out_hbm.at[:, pl.ds(base, _BPW)])

    return gather_kernel(tbl_flat, t)


def _tc_body(xtT_ref, scT_ref, l48_ref, out_ref):
    xtT = xtT_ref[...]
    sc = scT_ref[...]
    s_sum = jnp.sum(xtT, axis=0, keepdims=True)
    qs, qd = sc[0:1], sc[1:2]
    bs, bd = sc[2:3], sc[3:4]
    cs, cd = sc[4:5], sc[5:6]
    a = qd * s_sum + (qs - qd) * xtT
    p1 = cd * s_sum + (cs - cd) * xtT
    r = 1.0 / jnp.maximum(p1, 1e-5)
    u = bd * r
    w = (bs - bd) * r
    cat = jnp.concatenate([a, u, w], axis=0)
    res = lax.dot_general(
        l48_ref[...], cat, (((1,), (0,)), ((), ())),
        preferred_element_type=jnp.float32)
    out_ref[...] = res[:256] * res[256:]


def _tc_dense(xtT, scT, n):
    return pl.pallas_call(
        _tc_body,
        grid=(n // _B,),
        in_specs=[
            pl.BlockSpec((_K, _B), lambda i: (0, i)),
            pl.BlockSpec((8, _B), lambda i: (0, i)),
            pl.BlockSpec((512, 48), lambda i: (0, 0)),
        ],
        out_specs=pl.BlockSpec((256, _B), lambda i: (0, i)),
        out_shape=jax.ShapeDtypeStruct((256, n), jnp.float32),
    )(xtT, scT, jnp.asarray(_R48.T))


def kernel(xt, t, Qt, Qt_bar, Qt_bar_prev):
    n = xt.shape[0]
    tbl6 = jnp.stack(
        [Qt[:, 0, 0], Qt[:, 0, 1],
         Qt_bar_prev[:, 0, 0], Qt_bar_prev[:, 0, 1],
         Qt_bar[:, 0, 0], Qt_bar[:, 0, 1]], axis=1)
    tbl = jnp.zeros((8, _TPAD), jnp.float32).at[:6, :_T].set(tbl6.T)
    scT = _sc_gather(tbl.reshape(-1), t)
    xtT = xt.T
    out = _tc_dense(xtT, scT, n)
    return out.reshape(_K, _K, n).transpose(2, 0, 1)


# speedup vs baseline: 1.6562x; 1.2272x over previous
import functools

import jax
import jax.numpy as jnp
import numpy as np
from jax import lax
from jax.experimental import pallas as pl
from jax.experimental.pallas import tpu as pltpu
from jax.experimental.pallas import tpu_sc as plsc

_N = 262144
_K = 16
_T = 500
_TPAD = 512
_NC, _NS = 2, 16
_NW = _NC * _NS
_BPW = _N // _NW
_B = 4096


def _build_r48() -> np.ndarray:
    r = np.zeros((48, 512), np.float32)
    for i in range(_K):
        for k in range(_K):
            c = i * _K + k
            r[k, c] = 1.0
            r[16 + i, 256 + c] = 1.0
            if i == k:
                r[32 + i, 256 + c] = 1.0
    return r


_R48 = _build_r48()


def _sc_gather(tbl_flat, t):
    mesh = plsc.VectorSubcoreMesh(
        core_axis_name="c", subcore_axis_name="s",
        num_cores=_NC, num_subcores=_NS)

    @functools.partial(
        pl.kernel,
        out_type=jax.ShapeDtypeStruct((8, _N), jnp.float32),
        mesh=mesh,
        compiler_params=pltpu.CompilerParams(needs_layout_passes=False),
        scratch_types=[
            pltpu.VMEM((8 * _TPAD,), jnp.float32),
            pltpu.VMEM((_BPW,), jnp.int32),
            pltpu.VMEM((8, _BPW), jnp.float32),
        ],
    )
    def gather_kernel(tbl_hbm, t_hbm, out_hbm, tbl_v, idx_v, buf_v):
        wid = lax.axis_index("s") * _NC + lax.axis_index("c")
        base = wid * _BPW
        pltpu.sync_copy(tbl_hbm, tbl_v)
        pltpu.sync_copy(t_hbm.at[pl.ds(base, _BPW)], idx_v)

        def body(g, carry):
            off = g * 16
            tvec = idx_v[pl.ds(off, 16)]
            for c in range(8):
                v = plsc.load_gather(tbl_v, [tvec + (c * _TPAD)])
                buf_v[c, pl.ds(off, 16)] = v
            return carry

        lax.fori_loop(0, _BPW // 16, body, 0)
        pltpu.sync_copy(buf_v, out_hbm.at[:, pl.ds(base, _BPW)])

    return gather_kernel(tbl_flat, t)


def _tc_body(xtT_ref, scT_ref, l48_ref, out_ref):
    xtT = xtT_ref[...]
    sc = scT_ref[...]
    s_sum = jnp.sum(xtT, axis=0, keepdims=True)
    qs, qd = sc[0:1], sc[1:2]
    bs, bd = sc[2:3], sc[3:4]
    cs, cd = sc[4:5], sc[5:6]
    a = qd * s_sum + (qs - qd) * xtT
    p1 = cd * s_sum + (cs - cd) * xtT
    r = 1.0 / jnp.maximum(p1, 1e-5)
    u = bd * r
    w = (bs - bd) * r
    cat = jnp.concatenate([a, u, w], axis=0)
    res = lax.dot_general(
        l48_ref[...], cat, (((1,), (0,)), ((), ())),
        preferred_element_type=jnp.float32)
    out_ref[...] = res[:256] * res[256:]


def _tc_dense(xtT, scT, n):
    return pl.pallas_call(
        _tc_body,
        grid=(n // _B,),
        in_specs=[
            pl.BlockSpec((_K, _B), lambda i: (0, i)),
            pl.BlockSpec((8, _B), lambda i: (0, i)),
            pl.BlockSpec((512, 48), lambda i: (0, 0)),
        ],
        out_specs=pl.BlockSpec((256, _B), lambda i: (0, i)),
        out_shape=jax.ShapeDtypeStruct((256, n), jnp.float32),
    )(xtT, scT, jnp.asarray(_R48.T))


def kernel(xt, t, Qt, Qt_bar, Qt_bar_prev):
    n = xt.shape[0]
    tbl6 = jnp.stack(
        [Qt[:, 0, 0], Qt[:, 0, 1],
         Qt_bar_prev[:, 0, 0], Qt_bar_prev[:, 0, 1],
         Qt_bar[:, 0, 0], Qt_bar[:, 0, 1]], axis=1)
    tbl = jnp.zeros((8, _TPAD), jnp.float32).at[:6, :_T].set(tbl6.T)
    scT = _sc_gather(tbl.reshape(-1), t)
    xtT = xt.T
    out = _tc_dense(xtT, scT, n)
    return out.reshape(_K, _K, n).transpose(2, 0, 1)


# speedup vs baseline: 1.8813x; 1.1359x over previous
import functools

import jax
import jax.numpy as jnp
import numpy as np
from jax import lax
from jax.experimental import pallas as pl
from jax.experimental.pallas import tpu as pltpu
from jax.experimental.pallas import tpu_sc as plsc

_N = 262144
_K = 16
_T = 500
_TPAD = 512
_NC, _NS = 2, 16
_NW = _NC * _NS
_BPW = _N // _NW
_B = 8192


def _build_r48() -> np.ndarray:
    r = np.zeros((48, 512), np.float32)
    for i in range(_K):
        for k in range(_K):
            c = i * _K + k
            r[k, c] = 1.0
            r[16 + i, 256 + c] = 1.0
            if i == k:
                r[32 + i, 256 + c] = 1.0
    return r


_R48 = _build_r48()


def _sc_gather(tbl_flat, t):
    mesh = plsc.VectorSubcoreMesh(
        core_axis_name="c", subcore_axis_name="s",
        num_cores=_NC, num_subcores=_NS)

    @functools.partial(
        pl.kernel,
        out_type=jax.ShapeDtypeStruct((8, _N), jnp.float32),
        mesh=mesh,
        compiler_params=pltpu.CompilerParams(needs_layout_passes=False),
        scratch_types=[
            pltpu.VMEM((8 * _TPAD,), jnp.float32),
            pltpu.VMEM((_BPW,), jnp.int32),
            pltpu.VMEM((8, _BPW), jnp.float32),
        ],
    )
    def gather_kernel(tbl_hbm, t_hbm, out_hbm, tbl_v, idx_v, buf_v):
        wid = lax.axis_index("s") * _NC + lax.axis_index("c")
        base = wid * _BPW
        pltpu.sync_copy(tbl_hbm, tbl_v)
        pltpu.sync_copy(t_hbm.at[pl.ds(base, _BPW)], idx_v)

        def body(g, carry):
            off = g * 16
            tvec = idx_v[pl.ds(off, 16)]
            for c in range(8):
                v = plsc.load_gather(tbl_v, [tvec + (c * _TPAD)])
                buf_v[c, pl.ds(off, 16)] = v
            return carry

        lax.fori_loop(0, _BPW // 16, body, 0)
        pltpu.sync_copy(buf_v, out_hbm.at[:, pl.ds(base, _BPW)])

    return gather_kernel(tbl_flat, t)


def _tc_body(xtT_ref, scT_ref, l48_ref, out_ref):
    xtT = xtT_ref[...]
    sc = scT_ref[...]
    s_sum = jnp.sum(xtT, axis=0, keepdims=True)
    qs, qd = sc[0:1], sc[1:2]
    bs, bd = sc[2:3], sc[3:4]
    cs, cd = sc[4:5], sc[5:6]
    a = qd * s_sum + (qs - qd) * xtT
    p1 = cd * s_sum + (cs - cd) * xtT
    r = 1.0 / jnp.maximum(p1, 1e-5)
    u = bd * r
    w = (bs - bd) * r
    cat = jnp.concatenate([a, u, w], axis=0)
    res = lax.dot_general(
        l48_ref[...], cat, (((1,), (0,)), ((), ())),
        preferred_element_type=jnp.float32)
    out_ref[...] = res[:256] * res[256:]


def _tc_dense(xtT, scT, n):
    return pl.pallas_call(
        _tc_body,
        grid=(n // _B,),
        in_specs=[
            pl.BlockSpec((_K, _B), lambda i: (0, i)),
            pl.BlockSpec((8, _B), lambda i: (0, i)),
            pl.BlockSpec((512, 48), lambda i: (0, 0)),
        ],
        out_specs=pl.BlockSpec((256, _B), lambda i: (0, i)),
        out_shape=jax.ShapeDtypeStruct((256, n), jnp.float32),
    )(xtT, scT, jnp.asarray(_R48.T))


def kernel(xt, t, Qt, Qt_bar, Qt_bar_prev):
    n = xt.shape[0]
    tbl6 = jnp.stack(
        [Qt[:, 0, 0], Qt[:, 0, 1],
         Qt_bar_prev[:, 0, 0], Qt_bar_prev[:, 0, 1],
         Qt_bar[:, 0, 0], Qt_bar[:, 0, 1]], axis=1)
    tbl = jnp.zeros((8, _TPAD), jnp.float32).at[:6, :_T].set(tbl6.T)
    scT = _sc_gather(tbl.reshape(-1), t)
    xtT = xt.T
    out = _tc_dense(xtT, scT, n)
    return out.reshape(_K, _K, n).transpose(2, 0, 1)


# speedup vs baseline: 1.9430x; 1.0328x over previous
import functools

import jax
import jax.numpy as jnp
import numpy as np
from jax import lax
from jax.experimental import pallas as pl
from jax.experimental.pallas import tpu as pltpu
from jax.experimental.pallas import tpu_sc as plsc

_N = 262144
_K = 16
_T = 500
_TPAD = 512
_NC, _NS = 2, 16
_NW = _NC * _NS
_BPW = _N // _NW
_B = 16384


def _build_r48() -> np.ndarray:
    r = np.zeros((48, 512), np.float32)
    for i in range(_K):
        for k in range(_K):
            c = i * _K + k
            r[k, c] = 1.0
            r[16 + i, 256 + c] = 1.0
            if i == k:
                r[32 + i, 256 + c] = 1.0
    return r


_R48 = _build_r48()


def _sc_gather(tbl_flat, t):
    mesh = plsc.VectorSubcoreMesh(
        core_axis_name="c", subcore_axis_name="s",
        num_cores=_NC, num_subcores=_NS)

    @functools.partial(
        pl.kernel,
        out_type=jax.ShapeDtypeStruct((8, _N), jnp.float32),
        mesh=mesh,
        compiler_params=pltpu.CompilerParams(needs_layout_passes=False),
        scratch_types=[
            pltpu.VMEM((8 * _TPAD,), jnp.float32),
            pltpu.VMEM((_BPW,), jnp.int32),
            pltpu.VMEM((8, _BPW), jnp.float32),
        ],
    )
    def gather_kernel(tbl_hbm, t_hbm, out_hbm, tbl_v, idx_v, buf_v):
        wid = lax.axis_index("s") * _NC + lax.axis_index("c")
        base = wid * _BPW
        pltpu.sync_copy(tbl_hbm, tbl_v)
        pltpu.sync_copy(t_hbm.at[pl.ds(base, _BPW)], idx_v)

        def body(g, carry):
            off = g * 16
            tvec = idx_v[pl.ds(off, 16)]
            for c in range(8):
                v = plsc.load_gather(tbl_v, [tvec + (c * _TPAD)])
                buf_v[c, pl.ds(off, 16)] = v
            return carry

        lax.fori_loop(0, _BPW // 16, body, 0)
        pltpu.sync_copy(buf_v, out_hbm.at[:, pl.ds(base, _BPW)])

    return gather_kernel(tbl_flat, t)


def _tc_body(xtT_ref, scT_ref, l48_ref, out_ref):
    xtT = xtT_ref[...]
    sc = scT_ref[...]
    s_sum = jnp.sum(xtT, axis=0, keepdims=True)
    qs, qd = sc[0:1], sc[1:2]
    bs, bd = sc[2:3], sc[3:4]
    cs, cd = sc[4:5], sc[5:6]
    a = qd * s_sum + (qs - qd) * xtT
    p1 = cd * s_sum + (cs - cd) * xtT
    r = 1.0 / jnp.maximum(p1, 1e-5)
    u = bd * r
    w = (bs - bd) * r
    cat = jnp.concatenate([a, u, w], axis=0)
    res = lax.dot_general(
        l48_ref[...], cat, (((1,), (0,)), ((), ())),
        preferred_element_type=jnp.float32)
    out_ref[...] = res[:256] * res[256:]


def _tc_dense(xtT, scT, n):
    return pl.pallas_call(
        _tc_body,
        grid=(n // _B,),
        in_specs=[
            pl.BlockSpec((_K, _B), lambda i: (0, i)),
            pl.BlockSpec((8, _B), lambda i: (0, i)),
            pl.BlockSpec((512, 48), lambda i: (0, 0)),
        ],
        out_specs=pl.BlockSpec((256, _B), lambda i: (0, i)),
        out_shape=jax.ShapeDtypeStruct((256, n), jnp.float32),
    )(xtT, scT, jnp.asarray(_R48.T))


def kernel(xt, t, Qt, Qt_bar, Qt_bar_prev):
    n = xt.shape[0]
    tbl6 = jnp.stack(
        [Qt[:, 0, 0], Qt[:, 0, 1],
         Qt_bar_prev[:, 0, 0], Qt_bar_prev[:, 0, 1],
         Qt_bar[:, 0, 0], Qt_bar[:, 0, 1]], axis=1)
    tbl = jnp.zeros((8, _TPAD), jnp.float32).at[:6, :_T].set(tbl6.T)
    scT = _sc_gather(tbl.reshape(-1), t)
    xtT = xt.T
    out = _tc_dense(xtT, scT, n)
    return out.reshape(_K, _K, n).transpose(2, 0, 1)


# speedup vs baseline: 2.0096x; 1.0343x over previous
import functools

import jax
import jax.numpy as jnp
import numpy as np
from jax import lax
from jax.experimental import pallas as pl
from jax.experimental.pallas import tpu as pltpu
from jax.experimental.pallas import tpu_sc as plsc

_N = 262144
_K = 16
_T = 500
_TPAD = 512
_NC, _NS = 2, 16
_NW = _NC * _NS
_BPW = _N // _NW
_B = 16384


def _build_r48() -> np.ndarray:
    r = np.zeros((48, 512), np.float32)
    for i in range(_K):
        for k in range(_K):
            c = i * _K + k
            r[k, c] = 1.0
            r[16 + i, 256 + c] = 1.0
            if i == k:
                r[32 + i, 256 + c] = 1.0
    return r


_R48 = _build_r48()


def _sc_gather(tbl_flat, t):
    mesh = plsc.VectorSubcoreMesh(
        core_axis_name="c", subcore_axis_name="s",
        num_cores=_NC, num_subcores=_NS)

    @functools.partial(
        pl.kernel,
        out_type=jax.ShapeDtypeStruct((8, _N), jnp.float32),
        mesh=mesh,
        compiler_params=pltpu.CompilerParams(needs_layout_passes=False),
        scratch_types=[
            pltpu.VMEM((8 * _TPAD,), jnp.float32),
            pltpu.VMEM((_BPW,), jnp.int32),
            pltpu.VMEM((8, _BPW), jnp.float32),
        ],
    )
    def gather_kernel(tbl_hbm, t_hbm, out_hbm, tbl_v, idx_v, buf_v):
        wid = lax.axis_index("s") * _NC + lax.axis_index("c")
        base = wid * _BPW
        pltpu.sync_copy(tbl_hbm, tbl_v)
        pltpu.sync_copy(t_hbm.at[pl.ds(base, _BPW)], idx_v)

        unroll = 4

        def body(g, carry):
            for u in range(unroll):
                off = (g * unroll + u) * 16
                tvec = idx_v[pl.ds(off, 16)]
                for c in range(6):
                    v = plsc.load_gather(tbl_v, [tvec + (c * _TPAD)])
                    buf_v[c, pl.ds(off, 16)] = v
            return carry

        lax.fori_loop(0, _BPW // (16 * unroll), body, 0)
        pltpu.sync_copy(buf_v, out_hbm.at[:, pl.ds(base, _BPW)])

    return gather_kernel(tbl_flat, t)


def _tc_body(xtT_ref, scT_ref, l48_ref, out_ref):
    xtT = xtT_ref[...]
    sc = scT_ref[...]
    s_sum = jnp.sum(xtT, axis=0, keepdims=True)
    qs, qd = sc[0:1], sc[1:2]
    bs, bd = sc[2:3], sc[3:4]
    cs, cd = sc[4:5], sc[5:6]
    a = qd * s_sum + (qs - qd) * xtT
    p1 = cd * s_sum + (cs - cd) * xtT
    r = 1.0 / jnp.maximum(p1, 1e-5)
    u = bd * r
    w = (bs - bd) * r
    cat = jnp.concatenate([a, u, w], axis=0)
    res = lax.dot_general(
        l48_ref[...], cat, (((1,), (0,)), ((), ())),
        preferred_element_type=jnp.float32)
    out_ref[...] = res[:256] * res[256:]


def _tc_dense(xtT, scT, n):
    return pl.pallas_call(
        _tc_body,
        grid=(n // _B,),
        in_specs=[
            pl.BlockSpec((_K, _B), lambda i: (0, i)),
            pl.BlockSpec((8, _B), lambda i: (0, i)),
            pl.BlockSpec((512, 48), lambda i: (0, 0)),
        ],
        out_specs=pl.BlockSpec((256, _B), lambda i: (0, i)),
        out_shape=jax.ShapeDtypeStruct((256, n), jnp.float32),
    )(xtT, scT, jnp.asarray(_R48.T))


def kernel(xt, t, Qt, Qt_bar, Qt_bar_prev):
    n = xt.shape[0]
    tbl6 = jnp.stack(
        [Qt[:, 0, 0], Qt[:, 0, 1],
         Qt_bar_prev[:, 0, 0], Qt_bar_prev[:, 0, 1],
         Qt_bar[:, 0, 0], Qt_bar[:, 0, 1]], axis=1)
    tbl = jnp.zeros((8, _TPAD), jnp.float32).at[:6, :_T].set(tbl6.T)
    scT = _sc_gather(tbl.reshape(-1), t)
    xtT = xt.T
    out = _tc_dense(xtT, scT, n)
    return out.reshape(_K, _K, n).transpose(2, 0, 1)
